# baseline (speedup 1.0000x reference)
import functools

import jax
import jax.numpy as jnp
from jax import lax
from jax.experimental import pallas as pl
from jax.experimental.pallas import tpu as pltpu
from jax.experimental.pallas import tpu_sc as plsc

_NUM_FIELDS = 26
_VOCAB = 100000
_EMBED_DIM = 32
_BATCH = 16384

_NC, _NS = 2, 16
_NW = _NC * _NS
_NROWS = _NUM_FIELDS * _EMBED_DIM
_RPW = _NROWS // _NW
_OCHUNK = _BATCH // 4
_NCHUNK = _BATCH // _OCHUNK


def _mesh():
    return plsc.VectorSubcoreMesh(
        core_axis_name="c", subcore_axis_name="s",
        num_cores=_NC, num_subcores=_NS)


@functools.partial(
    pl.kernel,
    out_type=jax.ShapeDtypeStruct((_NROWS, _BATCH), jnp.float32),
    mesh=_mesh(),
    scratch_types=[
        pltpu.VMEM((_VOCAB,), jnp.float32),
        pltpu.VMEM((_BATCH,), jnp.int32),
        pltpu.VMEM((_OCHUNK,), jnp.float32),
        pltpu.VMEM((_OCHUNK,), jnp.float32),
        pltpu.SemaphoreType.DMA,
        pltpu.SemaphoreType.DMA,
        pltpu.SemaphoreType.DMA,
    ],
    compiler_params=pltpu.CompilerParams(
        use_tc_tiling_on_sc=True, needs_layout_passes=False),
)
def _embed_gather(x_hbm, tab_hbm, out_hbm, row_v, idx_v, out_a, out_b, wsem,
                  rsem, isem):
    wid = lax.axis_index("c") * _NS + lax.axis_index("s")
    outs = [out_a, out_b]

    def do_row(k, carry):
        r = wid + _NW * k
        field = k

        pltpu.async_copy(x_hbm.at[field], idx_v, isem)
        pltpu.async_copy(tab_hbm.at[r], row_v, rsem).wait()
        pltpu.make_async_copy(x_hbm.at[field], idx_v, isem).wait()

        for c in range(_NCHUNK):
            buf = outs[c % 2]
            dst = out_hbm.at[r, pl.ds(c * _OCHUNK, _OCHUNK)]

            pbuf = outs[(c - 1) % 2]
            if c == 0:
                @pl.when(k > 0)
                def _():
                    pdst = out_hbm.at[r - _NW,
                                      pl.ds((_NCHUNK - 1) * _OCHUNK, _OCHUNK)]
                    pltpu.async_copy(pbuf, pdst, wsem)
            else:
                pdst = out_hbm.at[r, pl.ds((c - 1) * _OCHUNK, _OCHUNK)]
                pltpu.async_copy(pbuf, pdst, wsem)

            drain = lambda: pltpu.make_async_copy(buf, dst, wsem).wait()
            if c < 2:
                pl.when(k > 0)(drain)
            else:
                drain()

            @plsc.parallel_loop(0, _OCHUNK // 16, unroll=16)
            def gather16(g):
                off = g * 16
                vals = plsc.load_gather(
                    row_v, [idx_v[pl.ds(c * _OCHUNK + off, 16)]])
                buf[pl.ds(off, 16)] = vals
        return carry

    lax.fori_loop(0, _RPW, do_row, jnp.int32(0))

    last_r = wid + _NW * (_RPW - 1)
    last_dst = out_hbm.at[last_r, pl.ds((_NCHUNK - 1) * _OCHUNK, _OCHUNK)]
    pltpu.async_copy(outs[(_NCHUNK - 1) % 2], last_dst, wsem)
    pltpu.make_async_copy(out_a, last_dst, wsem).wait()
    pltpu.make_async_copy(out_b, last_dst, wsem).wait()


def kernel(x, tables):
    x_t = x.astype(jnp.int32).T
    tab_t = tables.transpose(0, 2, 1).reshape(_NROWS, _VOCAB)
    out_t = _embed_gather(x_t, tab_t)
    return out_t.T.reshape(_BATCH, _NUM_FIELDS * _EMBED_DIM)


# speedup vs baseline: 1.2304x; 1.2304x over previous
import functools

import jax
import jax.numpy as jnp
from jax import lax
from jax.experimental import pallas as pl
from jax.experimental.pallas import tpu as pltpu
from jax.experimental.pallas import tpu_sc as plsc

_NUM_FIELDS = 26
_VOCAB = 100000
_EMBED_DIM = 32
_BATCH = 16384

_NC, _NS = 2, 16
_NW = _NC * _NS
_NROWS = _NUM_FIELDS * _EMBED_DIM
_RPW = _NROWS // _NW
_OCHUNK = _BATCH // 4
_NCHUNK = _BATCH // _OCHUNK


def _mesh():
    return plsc.VectorSubcoreMesh(
        core_axis_name="c", subcore_axis_name="s",
        num_cores=_NC, num_subcores=_NS)


@functools.partial(
    pl.kernel,
    out_type=jax.ShapeDtypeStruct((_NROWS, _BATCH), jnp.float32),
    mesh=_mesh(),
    scratch_types=[
        pltpu.VMEM((_VOCAB,), jnp.float32),
        pltpu.VMEM((_BATCH,), jnp.int32),
        pltpu.VMEM((_OCHUNK,), jnp.float32),
        pltpu.VMEM((_OCHUNK,), jnp.float32),
        pltpu.SemaphoreType.DMA,
        pltpu.SemaphoreType.DMA,
        pltpu.SemaphoreType.DMA,
    ],
    compiler_params=pltpu.CompilerParams(
        use_tc_tiling_on_sc=True, needs_layout_passes=False),
)
def _embed_gather(x_hbm, tab_hbm, out_hbm, row_v, idx_v, out_a, out_b, wsem,
                  rsem, isem):
    wid = lax.axis_index("c") * _NS + lax.axis_index("s")
    r0 = wid * _RPW
    outs = [out_a, out_b]

    def do_row(k, prev_field):
        r = r0 + k
        field = lax.shift_right_logical(r, 5)

        @pl.when(jnp.logical_or(k == 0, field != prev_field))
        def _():
            pltpu.async_copy(x_hbm.at[field], idx_v, isem).wait()

        pltpu.async_copy(tab_hbm.at[r], row_v, rsem).wait()

        for c in range(_NCHUNK):
            buf = outs[c % 2]
            dst = out_hbm.at[r, pl.ds(c * _OCHUNK, _OCHUNK)]

            pbuf = outs[(c - 1) % 2]
            if c == 0:
                @pl.when(k > 0)
                def _():
                    pdst = out_hbm.at[r - 1,
                                      pl.ds((_NCHUNK - 1) * _OCHUNK, _OCHUNK)]
                    pltpu.async_copy(pbuf, pdst, wsem)
            else:
                pdst = out_hbm.at[r, pl.ds((c - 1) * _OCHUNK, _OCHUNK)]
                pltpu.async_copy(pbuf, pdst, wsem)

            drain = lambda: pltpu.make_async_copy(buf, dst, wsem).wait()
            if c < 2:
                pl.when(k > 0)(drain)
            else:
                drain()

            @plsc.parallel_loop(0, _OCHUNK // 16, unroll=16)
            def gather16(g):
                off = g * 16
                vals = plsc.load_gather(
                    row_v, [idx_v[pl.ds(c * _OCHUNK + off, 16)]])
                buf[pl.ds(off, 16)] = vals
        return field

    lax.fori_loop(0, _RPW, do_row, jnp.int32(-1))

    last_r = r0 + _RPW - 1
    last_dst = out_hbm.at[last_r, pl.ds((_NCHUNK - 1) * _OCHUNK, _OCHUNK)]
    pltpu.async_copy(outs[(_NCHUNK - 1) % 2], last_dst, wsem)
    pltpu.make_async_copy(out_a, last_dst, wsem).wait()
    pltpu.make_async_copy(out_b, last_dst, wsem).wait()


def kernel(x, tables):
    x_t = x.astype(jnp.int32).T
    tab_t = tables.transpose(0, 2, 1).reshape(_NROWS, _VOCAB)
    out_t = _embed_gather(x_t, tab_t)
    return out_t.T.reshape(_BATCH, _NUM_FIELDS * _EMBED_DIM)
